# baseline (speedup 1.0000x reference)
import jax
import jax.numpy as jnp
from jax.experimental import pallas as pl
from jax.experimental.pallas import tpu as pltpu
from jax.experimental.pallas import tpu_sc as plsc

_B = 4
_T = 8192
_D = 1024


def _last_pool_body(x_hbm, leng_hbm, out_hbm, leng_s, sem):
    pltpu.sync_copy(leng_hbm, leng_s)
    copies = []
    for b in range(_B):
        l = leng_s[b]
        t = jnp.where(l < 1, _T - 1, l - 1)
        copies.append(pltpu.async_copy(x_hbm.at[b * _T + t], out_hbm.at[b], sem))
    for c in copies:
        c.wait()


def kernel(x, leng):
    xflat = x.reshape(_B * _T, _D)
    mesh = plsc.ScalarSubcoreMesh(axis_name="c", num_cores=1)
    f = pl.kernel(
        _last_pool_body,
        mesh=mesh,
        out_type=jax.ShapeDtypeStruct((_B, _D), jnp.float32),
        scratch_types=[
            pltpu.SMEM((_B,), jnp.int32),
            pltpu.SemaphoreType.DMA,
        ],
    )
    return f(xflat, leng.astype(jnp.int32))
